# Initial kernel scaffold; baseline (speedup 1.0000x reference)
import jax
import jax.numpy as jnp
from jax.experimental import pallas as pl


def kernel(g, z, w1, w2):
    raise NotImplementedError("write your pallas kernel here")


# speedup vs baseline: 2.2061x; 2.2061x over previous
import jax
import jax.numpy as jnp
from jax.experimental import pallas as pl
from jax.experimental.pallas import tpu as pltpu

_TM = 512


def _zw1_body(z_ref, w1t_ref, out_ref):
    out_ref[...] = jnp.dot(
        z_ref[...], w1t_ref[...], preferred_element_type=jnp.float32
    ).astype(out_ref.dtype)


def _layer1_body(g_ref, zw1_ref, w2t_ref, z1_ref, p_ref):
    y = jnp.dot(g_ref[...], zw1_ref[...], preferred_element_type=jnp.float32)
    z1 = jnp.maximum(y, 0.0)
    z1_ref[...] = z1
    p_ref[...] = jnp.dot(
        z1, w2t_ref[...], preferred_element_type=jnp.float32
    ).astype(p_ref.dtype)


def _layer2_body(g_ref, p_ref, res_ref):
    y = jnp.dot(g_ref[...], p_ref[...], preferred_element_type=jnp.float32)
    res_ref[...] = jax.nn.sigmoid(y)


def kernel(g, z, w1, w2):
    n = g.shape[0]
    hid = w1.shape[0]
    f_out = w2.shape[0]

    w1t = jnp.transpose(w1)
    w2t = jnp.transpose(w2)

    zw1 = pl.pallas_call(
        _zw1_body,
        out_shape=jax.ShapeDtypeStruct((n, hid), jnp.float32),
        in_specs=[
            pl.BlockSpec(memory_space=pltpu.MemorySpace.VMEM),
            pl.BlockSpec(memory_space=pltpu.MemorySpace.VMEM),
        ],
        out_specs=pl.BlockSpec(memory_space=pltpu.MemorySpace.VMEM),
    )(z, w1t)

    grid = (n // _TM,)

    z1, p = pl.pallas_call(
        _layer1_body,
        out_shape=(
            jax.ShapeDtypeStruct((n, hid), jnp.float32),
            jax.ShapeDtypeStruct((n, f_out), jnp.float32),
        ),
        grid=grid,
        in_specs=[
            pl.BlockSpec((_TM, n), lambda i: (i, 0)),
            pl.BlockSpec((n, hid), lambda i: (0, 0)),
            pl.BlockSpec((hid, f_out), lambda i: (0, 0)),
        ],
        out_specs=(
            pl.BlockSpec((_TM, hid), lambda i: (i, 0)),
            pl.BlockSpec((_TM, f_out), lambda i: (i, 0)),
        ),
        compiler_params=pltpu.CompilerParams(
            dimension_semantics=("parallel",),
            vmem_limit_bytes=60 * 1024 * 1024,
        ),
    )(g, zw1, w2t)

    res = pl.pallas_call(
        _layer2_body,
        out_shape=jax.ShapeDtypeStruct((n, f_out), jnp.float32),
        grid=grid,
        in_specs=[
            pl.BlockSpec((_TM, n), lambda i: (i, 0)),
            pl.BlockSpec((n, f_out), lambda i: (0, 0)),
        ],
        out_specs=pl.BlockSpec((_TM, f_out), lambda i: (i, 0)),
        compiler_params=pltpu.CompilerParams(
            dimension_semantics=("parallel",),
            vmem_limit_bytes=60 * 1024 * 1024,
        ),
    )(g, p)

    return res, z1


# speedup vs baseline: 2.2281x; 1.0099x over previous
import jax
import jax.numpy as jnp
from jax.experimental import pallas as pl
from jax.experimental.pallas import tpu as pltpu

_TM = 1024

_DN_T = (((1,), (1,)), ((), ()))


def _zw1_body(z_ref, w1_ref, out_ref):
    out_ref[...] = jax.lax.dot_general(
        z_ref[...], w1_ref[...], _DN_T, preferred_element_type=jnp.float32
    )


def _layer1_body(g_ref, zw1_ref, w2_ref, z1_ref, p_ref):
    y = jnp.dot(g_ref[...], zw1_ref[...], preferred_element_type=jnp.float32)
    z1 = jnp.maximum(y, 0.0)
    z1_ref[...] = z1
    p_ref[...] = jax.lax.dot_general(
        z1, w2_ref[...], _DN_T, preferred_element_type=jnp.float32
    )


def _layer2_body(g_ref, p_ref, res_ref):
    y = jnp.dot(g_ref[...], p_ref[...], preferred_element_type=jnp.float32)
    res_ref[...] = jax.nn.sigmoid(y)


def kernel(g, z, w1, w2):
    n = g.shape[0]
    hid = w1.shape[0]
    f_out = w2.shape[0]

    zw1 = pl.pallas_call(
        _zw1_body,
        out_shape=jax.ShapeDtypeStruct((n, hid), jnp.float32),
        in_specs=[
            pl.BlockSpec(memory_space=pltpu.MemorySpace.VMEM),
            pl.BlockSpec(memory_space=pltpu.MemorySpace.VMEM),
        ],
        out_specs=pl.BlockSpec(memory_space=pltpu.MemorySpace.VMEM),
    )(z, w1)

    grid = (n // _TM,)

    z1, p = pl.pallas_call(
        _layer1_body,
        out_shape=(
            jax.ShapeDtypeStruct((n, hid), jnp.float32),
            jax.ShapeDtypeStruct((n, f_out), jnp.float32),
        ),
        grid=grid,
        in_specs=[
            pl.BlockSpec((_TM, n), lambda i: (i, 0)),
            pl.BlockSpec((n, hid), lambda i: (0, 0)),
            pl.BlockSpec((f_out, hid), lambda i: (0, 0)),
        ],
        out_specs=(
            pl.BlockSpec((_TM, hid), lambda i: (i, 0)),
            pl.BlockSpec((_TM, f_out), lambda i: (i, 0)),
        ),
        compiler_params=pltpu.CompilerParams(
            dimension_semantics=("parallel",),
            vmem_limit_bytes=60 * 1024 * 1024,
        ),
    )(g, zw1, w2)

    res = pl.pallas_call(
        _layer2_body,
        out_shape=jax.ShapeDtypeStruct((n, f_out), jnp.float32),
        grid=grid,
        in_specs=[
            pl.BlockSpec((_TM, n), lambda i: (i, 0)),
            pl.BlockSpec((n, f_out), lambda i: (0, 0)),
        ],
        out_specs=pl.BlockSpec((_TM, f_out), lambda i: (i, 0)),
        compiler_params=pltpu.CompilerParams(
            dimension_semantics=("parallel",),
            vmem_limit_bytes=60 * 1024 * 1024,
        ),
    )(g, p)

    return res, z1


# speedup vs baseline: 2.2786x; 1.0227x over previous
import jax
import jax.numpy as jnp
from jax.experimental import pallas as pl
from jax.experimental.pallas import tpu as pltpu

_TM = 1024


def _layer1_body(g_ref, z_ref, w1t_ref, w2t_ref, z1_ref, p_ref):
    gz = jnp.dot(g_ref[...], z_ref[...], preferred_element_type=jnp.float32)
    z1 = jnp.maximum(
        jnp.dot(gz, w1t_ref[...], preferred_element_type=jnp.float32), 0.0
    )
    z1_ref[...] = z1
    p_ref[...] = jnp.dot(z1, w2t_ref[...], preferred_element_type=jnp.float32)


def _layer2_body(g_ref, p_ref, res_ref):
    y = jnp.dot(g_ref[...], p_ref[...], preferred_element_type=jnp.float32)
    res_ref[...] = jax.nn.sigmoid(y)


def kernel(g, z, w1, w2):
    n = g.shape[0]
    f_in = z.shape[1]
    hid = w1.shape[0]
    f_out = w2.shape[0]

    w1t = jnp.transpose(w1)
    w2t = jnp.transpose(w2)

    grid = (n // _TM,)

    z1, p = pl.pallas_call(
        _layer1_body,
        out_shape=(
            jax.ShapeDtypeStruct((n, hid), jnp.float32),
            jax.ShapeDtypeStruct((n, f_out), jnp.float32),
        ),
        grid=grid,
        in_specs=[
            pl.BlockSpec((_TM, n), lambda i: (i, 0)),
            pl.BlockSpec((n, f_in), lambda i: (0, 0)),
            pl.BlockSpec((f_in, hid), lambda i: (0, 0)),
            pl.BlockSpec((hid, f_out), lambda i: (0, 0)),
        ],
        out_specs=(
            pl.BlockSpec((_TM, hid), lambda i: (i, 0)),
            pl.BlockSpec((_TM, f_out), lambda i: (i, 0)),
        ),
        compiler_params=pltpu.CompilerParams(
            dimension_semantics=("parallel",),
            vmem_limit_bytes=60 * 1024 * 1024,
        ),
    )(g, z, w1t, w2t)

    res = pl.pallas_call(
        _layer2_body,
        out_shape=jax.ShapeDtypeStruct((n, f_out), jnp.float32),
        grid=grid,
        in_specs=[
            pl.BlockSpec((_TM, n), lambda i: (i, 0)),
            pl.BlockSpec((n, f_out), lambda i: (0, 0)),
        ],
        out_specs=pl.BlockSpec((_TM, f_out), lambda i: (i, 0)),
        compiler_params=pltpu.CompilerParams(
            dimension_semantics=("parallel",),
            vmem_limit_bytes=60 * 1024 * 1024,
        ),
    )(g, p)

    return res, z1
